# baseline (speedup 1.0000x reference)
import functools

import jax
import jax.numpy as jnp
from jax import lax
from jax.experimental import pallas as pl
from jax.experimental.pallas import tpu as pltpu

_THRESH = 0.7
_MIN_KEPT = 100000

_B, _C, _H, _W = 4, 19, 512, 512
_RH = 32
_NCH = _H // _RH


def _pass_body(*refs):
    xs = refs[:5]
    tgt_ref, thr_ref = refs[5], refs[6]
    cnt_ref, sumf_ref = refs[7], refs[8]
    b = pl.program_id(0)
    c = pl.program_id(1)

    @pl.when((b == 0) & (c == 0))
    def _init():
        cnt_ref[...] = jnp.zeros_like(cnt_ref)
        sumf_ref[...] = jnp.zeros_like(sumf_ref)

    nch = _RH // 8

    def tail(i, s_, xt_, shift, cp, fp):
        pg = jnp.exp(xt_ - shift) / s_
        logp = jnp.log(s_) - (xt_ - shift)
        f = (1.0 - pg) * (1.0 - pg) * logp
        keep = pg < thr_ref[i, 0]
        cp = cp + jnp.where(keep, 1.0, 0.0)
        fp = fp + jnp.where(keep, f, 0.0)
        return cp, fp

    zero = jnp.zeros((8, _W), jnp.float32)
    cps = [zero] * 5
    fps = [zero] * 5
    smx = [zero] * 5
    smn = [jnp.full((8, _W), 3.0e38, jnp.float32)] * 5
    for j in range(nch):
        rows = pl.ds(j * 8, 8)
        t8 = tgt_ref[0, rows, :]
        s = [jnp.exp(xs[i][0, 0, rows, :]) for i in range(5)]
        xt = [xs[i][0, 0, rows, :] for i in range(5)]
        for c_ in range(1, _C):
            sel = t8 == c_
            for i in range(5):
                v = xs[i][0, c_, rows, :]
                s[i] = s[i] + jnp.exp(v)
                xt[i] = jnp.where(sel, v, xt[i])
        for i in range(5):
            smx[i] = jnp.maximum(smx[i], s[i])
            smn[i] = jnp.minimum(smn[i], s[i])
            cps[i], fps[i] = tail(i, s[i], xt[i], 0.0, cps[i], fps[i])

    for i in range(5):
        safe = (jnp.max(smx[i]) < 3.0e38) & (jnp.min(smn[i]) > 1.0e-30)

        @pl.when(safe)
        def _commit_fast(i=i):
            cnt_ref[i] = cnt_ref[i] + cps[i]
            sumf_ref[i] = sumf_ref[i] + fps[i]

        @pl.when(jnp.logical_not(safe))
        def _redo_shifted(i=i):
            cp = jnp.zeros((8, _W), jnp.float32)
            fp = jnp.zeros((8, _W), jnp.float32)
            for j in range(nch):
                rows = pl.ds(j * 8, 8)
                t8 = tgt_ref[0, rows, :]
                m = xs[i][0, 0, rows, :]
                for c_ in range(1, _C):
                    m = jnp.maximum(m, xs[i][0, c_, rows, :])
                x0 = xs[i][0, 0, rows, :]
                s_ = jnp.exp(x0 - m)
                xt_ = x0
                for c_ in range(1, _C):
                    v = xs[i][0, c_, rows, :]
                    s_ = s_ + jnp.exp(v - m)
                    xt_ = jnp.where(t8 == c_, v, xt_)
                cp, fp = tail(i, s_, xt_, m, cp, fp)
            cnt_ref[i] = cnt_ref[i] + cp
            sumf_ref[i] = sumf_ref[i] + fp


def _masked_stats(xs, tgt, thr):
    grid = (_B, _NCH)
    x_spec = pl.BlockSpec((1, _C, _RH, _W), lambda b, c: (b, 0, c, 0))
    t_spec = pl.BlockSpec((1, _RH, _W), lambda b, c: (b, c, 0))
    thr_spec = pl.BlockSpec(memory_space=pltpu.SMEM)
    acc_spec = pl.BlockSpec((5, 8, _W), lambda b, c: (0, 0, 0))
    cnt, sumf = pl.pallas_call(
        _pass_body,
        grid=grid,
        in_specs=[x_spec] * 5 + [t_spec, thr_spec],
        out_specs=[acc_spec, acc_spec],
        out_shape=[
            jax.ShapeDtypeStruct((5, 8, _W), jnp.float32),
            jax.ShapeDtypeStruct((5, 8, _W), jnp.float32),
        ],
    )(*xs, tgt, thr)
    return jnp.sum(cnt, axis=(1, 2)), jnp.sum(sumf, axis=(1, 2))


def kernel(x1, x2, x3, x4, x5, target):
    xs = (x1, x2, x3, x4, x5)
    tgt = target.astype(jnp.int32)
    n = _B * _H * _W
    k = min(_MIN_KEPT, n - 1)

    thr0 = jnp.full((5, 1), _THRESH, jnp.float32)
    cnt0, sumf0 = _masked_stats(xs, tgt, thr0)

    def common():
        return jnp.sum(sumf0 / cnt0)

    def rare():
        lo = jnp.zeros((5,), jnp.int32)
        hi = jnp.full((5,), 0x3F800010, jnp.int32)

        def step(_, carry):
            lo_, hi_ = carry
            mid = (lo_ + hi_) >> 1
            t = lax.bitcast_convert_type(mid + 1, jnp.float32).reshape(5, 1)
            cnt, _unused = _masked_stats(xs, tgt, t)
            pred = cnt >= jnp.float32(k + 1)
            return jnp.where(pred, lo_, mid + 1), jnp.where(pred, mid, hi_)

        lo_f, _hi_f = lax.fori_loop(0, 31, step, (lo, hi))
        v = lax.bitcast_convert_type(lo_f, jnp.float32)
        thr = jnp.maximum(v, jnp.float32(_THRESH)).reshape(5, 1)
        cnt, sumf = _masked_stats(xs, tgt, thr)
        return jnp.sum(sumf / cnt)

    need_rare = jnp.any(cnt0 <= jnp.float32(k))
    return lax.cond(need_rare, rare, common)


# speedup vs baseline: 1.1299x; 1.1299x over previous
import functools

import jax
import jax.numpy as jnp
from jax import lax
from jax.experimental import pallas as pl
from jax.experimental.pallas import tpu as pltpu

_THRESH = 0.7
_MIN_KEPT = 100000

_B, _C, _H, _W = 4, 19, 512, 512
_RH = 64
_NCH = _H // _RH


def _pass_body(*refs):
    xs = refs[:5]
    tgt_ref, thr_ref = refs[5], refs[6]
    cnt_ref, sumf_ref = refs[7], refs[8]
    b = pl.program_id(0)
    c = pl.program_id(1)

    @pl.when((b == 0) & (c == 0))
    def _init():
        cnt_ref[...] = jnp.zeros_like(cnt_ref)
        sumf_ref[...] = jnp.zeros_like(sumf_ref)

    caccs = [cnt_ref[i] for i in range(5)]
    faccs = [sumf_ref[i] for i in range(5)]
    for j in range(_RH // 8):
        rows = pl.ds(j * 8, 8)
        t8 = tgt_ref[0, rows, :]
        m = [xs[i][0, 0, rows, :] for i in range(5)]
        for c_ in range(1, _C):
            for i in range(5):
                m[i] = jnp.maximum(m[i], xs[i][0, c_, rows, :])
        x0 = [xs[i][0, 0, rows, :] for i in range(5)]
        s = [jnp.exp(x0[i] - m[i]) for i in range(5)]
        xt = x0
        for c_ in range(1, _C):
            sel = t8 == c_
            for i in range(5):
                v = xs[i][0, c_, rows, :]
                s[i] = s[i] + jnp.exp(v - m[i])
                xt[i] = jnp.where(sel, v, xt[i])
        for i in range(5):
            d = xt[i] - m[i]
            pg = jnp.exp(d) / s[i]
            logp = jnp.log(s[i]) - d
            f = (1.0 - pg) * (1.0 - pg) * logp
            keep = pg < thr_ref[i, 0]
            caccs[i] = caccs[i] + jnp.where(keep, 1.0, 0.0)
            faccs[i] = faccs[i] + jnp.where(keep, f, 0.0)
    for i in range(5):
        cnt_ref[i] = caccs[i]
        sumf_ref[i] = faccs[i]


def _masked_stats(xs, tgt, thr):
    grid = (_B, _NCH)
    x_spec = pl.BlockSpec((1, _C, _RH, _W), lambda b, c: (b, 0, c, 0))
    t_spec = pl.BlockSpec((1, _RH, _W), lambda b, c: (b, c, 0))
    thr_spec = pl.BlockSpec(memory_space=pltpu.SMEM)
    acc_spec = pl.BlockSpec((5, 8, _W), lambda b, c: (0, 0, 0))
    cnt, sumf = pl.pallas_call(
        _pass_body,
        grid=grid,
        in_specs=[x_spec] * 5 + [t_spec, thr_spec],
        out_specs=[acc_spec, acc_spec],
        out_shape=[
            jax.ShapeDtypeStruct((5, 8, _W), jnp.float32),
            jax.ShapeDtypeStruct((5, 8, _W), jnp.float32),
        ],
    )(*xs, tgt, thr)
    return jnp.sum(cnt, axis=(1, 2)), jnp.sum(sumf, axis=(1, 2))


def kernel(x1, x2, x3, x4, x5, target):
    xs = (x1, x2, x3, x4, x5)
    tgt = target.astype(jnp.int32)
    n = _B * _H * _W
    k = min(_MIN_KEPT, n - 1)

    thr0 = jnp.full((5, 1), _THRESH, jnp.float32)
    cnt0, sumf0 = _masked_stats(xs, tgt, thr0)

    def common():
        return jnp.sum(sumf0 / cnt0)

    def rare():
        lo = jnp.zeros((5,), jnp.int32)
        hi = jnp.full((5,), 0x3F800010, jnp.int32)

        def step(_, carry):
            lo_, hi_ = carry
            mid = (lo_ + hi_) >> 1
            t = lax.bitcast_convert_type(mid + 1, jnp.float32).reshape(5, 1)
            cnt, _unused = _masked_stats(xs, tgt, t)
            pred = cnt >= jnp.float32(k + 1)
            return jnp.where(pred, lo_, mid + 1), jnp.where(pred, mid, hi_)

        lo_f, _hi_f = lax.fori_loop(0, 31, step, (lo, hi))
        v = lax.bitcast_convert_type(lo_f, jnp.float32)
        thr = jnp.maximum(v, jnp.float32(_THRESH)).reshape(5, 1)
        cnt, sumf = _masked_stats(xs, tgt, thr)
        return jnp.sum(sumf / cnt)

    need_rare = jnp.any(cnt0 <= jnp.float32(k))
    return lax.cond(need_rare, rare, common)


# speedup vs baseline: 1.1842x; 1.0480x over previous
import functools

import jax
import jax.numpy as jnp
from jax import lax
from jax.experimental import pallas as pl
from jax.experimental.pallas import tpu as pltpu

_THRESH = 0.7
_MIN_KEPT = 100000

_B, _C, _H, _W = 4, 19, 512, 512
_RH = 128
_NCH = _H // _RH


def _pass_body(*refs):
    xs = refs[:5]
    tgt_ref, thr_ref = refs[5], refs[6]
    cnt_ref, sumf_ref = refs[7], refs[8]
    b = pl.program_id(0)
    c = pl.program_id(1)

    @pl.when((b == 0) & (c == 0))
    def _init():
        cnt_ref[...] = jnp.zeros_like(cnt_ref)
        sumf_ref[...] = jnp.zeros_like(sumf_ref)

    caccs = [cnt_ref[i] for i in range(5)]
    faccs = [sumf_ref[i] for i in range(5)]
    for j in range(_RH // 8):
        rows = pl.ds(j * 8, 8)
        t8 = tgt_ref[0, rows, :]
        m = [xs[i][0, 0, rows, :] for i in range(5)]
        for c_ in range(1, _C):
            for i in range(5):
                m[i] = jnp.maximum(m[i], xs[i][0, c_, rows, :])
        x0 = [xs[i][0, 0, rows, :] for i in range(5)]
        s = [jnp.exp(x0[i] - m[i]) for i in range(5)]
        xt = x0
        for c_ in range(1, _C):
            sel = t8 == c_
            for i in range(5):
                v = xs[i][0, c_, rows, :]
                s[i] = s[i] + jnp.exp(v - m[i])
                xt[i] = jnp.where(sel, v, xt[i])
        for i in range(5):
            d = xt[i] - m[i]
            pg = jnp.exp(d) / s[i]
            logp = jnp.log(s[i]) - d
            f = (1.0 - pg) * (1.0 - pg) * logp
            keep = pg < thr_ref[i, 0]
            caccs[i] = caccs[i] + jnp.where(keep, 1.0, 0.0)
            faccs[i] = faccs[i] + jnp.where(keep, f, 0.0)
    for i in range(5):
        cnt_ref[i] = caccs[i]
        sumf_ref[i] = faccs[i]


def _masked_stats(xs, tgt, thr):
    grid = (_B, _NCH)
    x_spec = pl.BlockSpec((1, _C, _RH, _W), lambda b, c: (b, 0, c, 0))
    t_spec = pl.BlockSpec((1, _RH, _W), lambda b, c: (b, c, 0))
    thr_spec = pl.BlockSpec(memory_space=pltpu.SMEM)
    acc_spec = pl.BlockSpec((5, 8, _W), lambda b, c: (0, 0, 0))
    cnt, sumf = pl.pallas_call(
        _pass_body,
        grid=grid,
        in_specs=[x_spec] * 5 + [t_spec, thr_spec],
        out_specs=[acc_spec, acc_spec],
        out_shape=[
            jax.ShapeDtypeStruct((5, 8, _W), jnp.float32),
            jax.ShapeDtypeStruct((5, 8, _W), jnp.float32),
        ],
    )(*xs, tgt, thr)
    return jnp.sum(cnt, axis=(1, 2)), jnp.sum(sumf, axis=(1, 2))


def kernel(x1, x2, x3, x4, x5, target):
    xs = (x1, x2, x3, x4, x5)
    tgt = target.astype(jnp.int32)
    n = _B * _H * _W
    k = min(_MIN_KEPT, n - 1)

    thr0 = jnp.full((5, 1), _THRESH, jnp.float32)
    cnt0, sumf0 = _masked_stats(xs, tgt, thr0)

    def common():
        return jnp.sum(sumf0 / cnt0)

    def rare():
        lo = jnp.zeros((5,), jnp.int32)
        hi = jnp.full((5,), 0x3F800010, jnp.int32)

        def step(_, carry):
            lo_, hi_ = carry
            mid = (lo_ + hi_) >> 1
            t = lax.bitcast_convert_type(mid + 1, jnp.float32).reshape(5, 1)
            cnt, _unused = _masked_stats(xs, tgt, t)
            pred = cnt >= jnp.float32(k + 1)
            return jnp.where(pred, lo_, mid + 1), jnp.where(pred, mid, hi_)

        lo_f, _hi_f = lax.fori_loop(0, 31, step, (lo, hi))
        v = lax.bitcast_convert_type(lo_f, jnp.float32)
        thr = jnp.maximum(v, jnp.float32(_THRESH)).reshape(5, 1)
        cnt, sumf = _masked_stats(xs, tgt, thr)
        return jnp.sum(sumf / cnt)

    need_rare = jnp.any(cnt0 <= jnp.float32(k))
    return lax.cond(need_rare, rare, common)
